# baseline (speedup 1.0000x reference)
import functools

import jax
import jax.numpy as jnp
from jax import lax
from jax.experimental import pallas as pl
from jax.experimental.pallas import tpu as pltpu
from jax.experimental.pallas import tpu_sc as plsc

N_NODES = 10000
N_EDGES = 320000
D = 128
NC = 2
NS = 16
L = 16
NW = NC * NS
E_PER_W = N_EDGES // NW
CHUNK = 20
CHUNKS_PER_W = E_PER_W // CHUNK
NBUF = 6
ROWS_PER_TILE = N_NODES // NS


def _zero_rows(ref, n_rows, n_cols):
    per_row = n_cols // L

    def body(i, _):
        ref[i // per_row, pl.ds((i % per_row) * L, L)] = jnp.zeros((L,), jnp.float32)
        return 0

    lax.fori_loop(0, n_rows * per_row, body, 0)


def _agg_body(x_hbm, src2_hbm, dst2_hbm, p_hbm, deg_hbm,
              agg_sh, src_v, dst_v, rows, deg_v, gsems, ssems):
    cid = lax.axis_index("c")
    sid = lax.axis_index("s")
    wid = sid * NC + cid

    idx_src_cp = pltpu.async_copy(src2_hbm.at[wid], src_v, gsems.at[0])
    idx_dst_cp = pltpu.async_copy(dst2_hbm.at[wid], dst_v, gsems.at[1])

    _zero_rows(rows.at[0], CHUNK, D)

    zero_cps = []
    for t in range(ROWS_PER_TILE // CHUNK):
        zero_cps.append(pltpu.async_copy(
            rows.at[0],
            agg_sh.at[pl.ds(sid * ROWS_PER_TILE + t * CHUNK, CHUNK)],
            ssems.at[0]))
    rem = ROWS_PER_TILE % CHUNK
    if rem:
        zero_cps.append(pltpu.async_copy(
            rows.at[0, pl.ds(0, rem)],
            agg_sh.at[pl.ds(sid * ROWS_PER_TILE + (ROWS_PER_TILE // CHUNK) * CHUNK, rem)],
            ssems.at[0]))

    def zero_deg(i, _):
        deg_v[pl.ds(i * L, L)] = jnp.zeros((L,), jnp.float32)
        return 0

    lax.fori_loop(0, N_NODES // L, zero_deg, 0)

    idx_src_cp.wait()
    idx_dst_cp.wait()

    ones = jnp.full((L,), 1.0, jnp.float32)
    tailmask = lax.broadcasted_iota(jnp.int32, (L,), 0) >= (L - CHUNK % L)

    def gat(k, b):
        pltpu.async_copy(x_hbm.at[src_v.at[k]], rows.at[b], gsems.at[b])

    def gat_wait(k, b):
        pltpu.make_async_copy(x_hbm.at[src_v.at[k]], rows.at[b], gsems.at[b]).wait()

    def sca(k, b):
        pltpu.async_copy(rows.at[b], agg_sh.at[dst_v.at[k]], ssems.at[b], add=True)

    def sca_wait(k, b):
        pltpu.make_async_copy(rows.at[b], agg_sh.at[dst_v.at[k]], ssems.at[b]).wait()

    def deg_update(k):
        for j in range(CHUNK // L):
            idx = dst_v[k, pl.ds(j * L, L)]
            plsc.addupdate_scatter(deg_v, [idx], ones)
        if CHUNK % L:
            idx = dst_v[k, pl.ds(CHUNK - L, L)]
            plsc.addupdate_scatter(deg_v, [idx], ones, mask=tailmask)

    for b in range(1, NBUF):
        gat(b, b)
    for cp in zero_cps:
        cp.wait()
    gat(0, 0)

    plsc.subcore_barrier()

    def chunk_step(k, b):
        gat_wait(k, b)
        sca(k, b)
        deg_update(k)
        bp = (b - 1) % NBUF

        @pl.when(k >= 1)
        def _():
            sca_wait(k - 1, bp)

            @pl.when(k - 1 + NBUF < CHUNKS_PER_W)
            def _():
                gat(k - 1 + NBUF, bp)

    n_groups = CHUNKS_PER_W // NBUF
    def body(i, _):
        for b in range(NBUF):
            chunk_step(i * NBUF + b, b)
        return 0

    lax.fori_loop(0, n_groups, body, 0)

    for k in range(n_groups * NBUF, CHUNKS_PER_W):
        chunk_step(k, k % NBUF)

    last = CHUNKS_PER_W - 1
    sca_wait(last, last % NBUF)

    plsc.subcore_barrier()

    row0 = sid * ROWS_PER_TILE
    pltpu.sync_copy(agg_sh.at[pl.ds(row0, ROWS_PER_TILE)],
                    p_hbm.at[cid, sid])
    pltpu.sync_copy(deg_v, deg_hbm.at[wid, 0])


_agg = pl.kernel(
    _agg_body,
    out_type=(
        jax.ShapeDtypeStruct((NC, NS, ROWS_PER_TILE, D), jnp.float32),
        jax.ShapeDtypeStruct((NW, 1, N_NODES), jnp.float32),
    ),
    mesh=plsc.VectorSubcoreMesh(core_axis_name="c", subcore_axis_name="s"),
    compiler_params=pltpu.CompilerParams(
        needs_layout_passes=False, use_tc_tiling_on_sc=False),
    scratch_types=[
        pltpu.VMEM_SHARED((N_NODES, D), jnp.float32),
        pltpu.VMEM((CHUNKS_PER_W, CHUNK), jnp.int32),
        pltpu.VMEM((CHUNKS_PER_W, CHUNK), jnp.int32),
        pltpu.VMEM((NBUF, CHUNK, D), jnp.float32),
        pltpu.VMEM((N_NODES,), jnp.float32),
        pltpu.SemaphoreType.DMA((NBUF,)),
        pltpu.SemaphoreType.DMA((NBUF,)),
    ],
)


def _dense_body(x_ref, p_ref, deg_ref, wst_ref, wnt_ref, bg_ref,
                wzt_ref, wxt_ref, bo_ref, o_ref):
    x = x_ref[...]
    deg = jnp.sum(deg_ref[...], axis=0)
    scale = 1.0 / jnp.maximum(deg, 1.0)
    agg = (p_ref[0] + p_ref[1]) * scale[:, None]
    z = jnp.dot(x, wst_ref[...], preferred_element_type=jnp.float32)
    z = z + jnp.dot(agg, wnt_ref[...], preferred_element_type=jnp.float32)
    z = jnp.maximum(z + bg_ref[...], 0.0)
    o = jnp.dot(z, wzt_ref[...], preferred_element_type=jnp.float32)
    o = o + jnp.dot(x, wxt_ref[...], preferred_element_type=jnp.float32)
    o_ref[...] = o + bo_ref[...]


@functools.partial(jax.jit, static_argnums=())
def _dense(x, p, deg, wst, wnt, bg, wzt, wxt, bo):
    return pl.pallas_call(
        _dense_body,
        out_shape=jax.ShapeDtypeStruct((N_NODES, D), jnp.float32),
    )(x, p, deg, wst, wnt, bg, wzt, wxt, bo)


def kernel(x, edge_index, W_self, W_nbr, b_gnn, W_out, b_out):
    src = edge_index[0].astype(jnp.int32)
    dst = edge_index[1].astype(jnp.int32)
    src2 = src.reshape(NW, CHUNKS_PER_W, CHUNK)
    dst2 = dst.reshape(NW, CHUNKS_PER_W, CHUNK)
    p, deg = _agg(x, src2, dst2)
    p = p.reshape(NC, N_NODES, D)
    deg = deg.reshape(NW, N_NODES)
    return _dense(
        x, p, deg,
        W_self.T, W_nbr.T, b_gnn.reshape(1, D),
        W_out[:, :D].T, W_out[:, D:].T, b_out.reshape(1, D),
    )


# speedup vs baseline: 1.3320x; 1.3320x over previous
import functools

import jax
import jax.numpy as jnp
from jax import lax
from jax.experimental import pallas as pl
from jax.experimental.pallas import tpu as pltpu
from jax.experimental.pallas import tpu_sc as plsc

N_NODES = 10000
N_EDGES = 320000
D = 128
NC = 2
NS = 16
L = 16
NW = NC * NS
E_PER_W = N_EDGES // NW
CHUNK = 40
CHUNKS_PER_W = E_PER_W // CHUNK
NBUF = 4
ROWS_PER_TILE = N_NODES // NS


def _zero_rows(ref, n_rows, n_cols):
    per_row = n_cols // L

    def body(i, _):
        ref[i // per_row, pl.ds((i % per_row) * L, L)] = jnp.zeros((L,), jnp.float32)
        return 0

    lax.fori_loop(0, n_rows * per_row, body, 0)


def _agg_body(x_hbm, src2_hbm, dst2_hbm, p_hbm, deg_hbm,
              agg_sh, src_v, dst_v, rows, deg_v, gsems, ssems):
    cid = lax.axis_index("c")
    sid = lax.axis_index("s")
    wid = sid * NC + cid

    idx_src_cp = pltpu.async_copy(src2_hbm.at[wid], src_v, gsems.at[0])
    idx_dst_cp = pltpu.async_copy(dst2_hbm.at[wid], dst_v, gsems.at[1])

    _zero_rows(rows.at[0], CHUNK, D)

    zero_cps = []
    for t in range(ROWS_PER_TILE // CHUNK):
        zero_cps.append(pltpu.async_copy(
            rows.at[0],
            agg_sh.at[pl.ds(sid * ROWS_PER_TILE + t * CHUNK, CHUNK)],
            ssems.at[0]))
    rem = ROWS_PER_TILE % CHUNK
    if rem:
        zero_cps.append(pltpu.async_copy(
            rows.at[0, pl.ds(0, rem)],
            agg_sh.at[pl.ds(sid * ROWS_PER_TILE + (ROWS_PER_TILE // CHUNK) * CHUNK, rem)],
            ssems.at[0]))

    def zero_deg(i, _):
        deg_v[pl.ds(i * L, L)] = jnp.zeros((L,), jnp.float32)
        return 0

    lax.fori_loop(0, N_NODES // L, zero_deg, 0)

    idx_src_cp.wait()
    idx_dst_cp.wait()

    ones = jnp.full((L,), 1.0, jnp.float32)
    tailmask = lax.broadcasted_iota(jnp.int32, (L,), 0) >= (L - CHUNK % L)

    def gat(k, b):
        pltpu.async_copy(x_hbm.at[src_v.at[k]], rows.at[b], gsems.at[b])

    def gat_wait(k, b):
        pltpu.make_async_copy(x_hbm.at[src_v.at[k]], rows.at[b], gsems.at[b]).wait()

    def sca(k, b):
        pltpu.async_copy(rows.at[b], agg_sh.at[dst_v.at[k]], ssems.at[b], add=True)

    def sca_wait(k, b):
        pltpu.make_async_copy(rows.at[b], agg_sh.at[dst_v.at[k]], ssems.at[b]).wait()

    def deg_update(k):
        for j in range(CHUNK // L):
            idx = dst_v[k, pl.ds(j * L, L)]
            plsc.addupdate_scatter(deg_v, [idx], ones)
        if CHUNK % L:
            idx = dst_v[k, pl.ds(CHUNK - L, L)]
            plsc.addupdate_scatter(deg_v, [idx], ones, mask=tailmask)

    for b in range(1, NBUF):
        gat(b, b)
    for cp in zero_cps:
        cp.wait()
    gat(0, 0)

    plsc.subcore_barrier()

    def chunk_step(k, b):
        gat_wait(k, b)
        sca(k, b)
        deg_update(k)
        bp = (b - 1) % NBUF

        @pl.when(k >= 1)
        def _():
            sca_wait(k - 1, bp)

            @pl.when(k - 1 + NBUF < CHUNKS_PER_W)
            def _():
                gat(k - 1 + NBUF, bp)

    n_groups = CHUNKS_PER_W // NBUF
    def body(i, _):
        for b in range(NBUF):
            chunk_step(i * NBUF + b, b)
        return 0

    lax.fori_loop(0, n_groups, body, 0)

    for k in range(n_groups * NBUF, CHUNKS_PER_W):
        chunk_step(k, k % NBUF)

    last = CHUNKS_PER_W - 1
    sca_wait(last, last % NBUF)

    plsc.subcore_barrier()

    row0 = sid * ROWS_PER_TILE
    pltpu.sync_copy(agg_sh.at[pl.ds(row0, ROWS_PER_TILE)],
                    p_hbm.at[cid, sid])
    pltpu.sync_copy(deg_v, deg_hbm.at[wid, 0])


_agg = pl.kernel(
    _agg_body,
    out_type=(
        jax.ShapeDtypeStruct((NC, NS, ROWS_PER_TILE, D), jnp.float32),
        jax.ShapeDtypeStruct((NW, 1, N_NODES), jnp.float32),
    ),
    mesh=plsc.VectorSubcoreMesh(core_axis_name="c", subcore_axis_name="s"),
    compiler_params=pltpu.CompilerParams(
        needs_layout_passes=False, use_tc_tiling_on_sc=False),
    scratch_types=[
        pltpu.VMEM_SHARED((N_NODES, D), jnp.float32),
        pltpu.VMEM((CHUNKS_PER_W, CHUNK), jnp.int32),
        pltpu.VMEM((CHUNKS_PER_W, CHUNK), jnp.int32),
        pltpu.VMEM((NBUF, CHUNK, D), jnp.float32),
        pltpu.VMEM((N_NODES,), jnp.float32),
        pltpu.SemaphoreType.DMA((NBUF,)),
        pltpu.SemaphoreType.DMA((NBUF,)),
    ],
)


def _dense_body(x_ref, p_ref, deg_ref, wst_ref, wnt_ref, bg_ref,
                wzt_ref, wxt_ref, bo_ref, o_ref):
    x = x_ref[...]
    deg = jnp.sum(deg_ref[...], axis=0)
    scale = 1.0 / jnp.maximum(deg, 1.0)
    agg = (p_ref[0] + p_ref[1]) * scale[:, None]
    z = jnp.dot(x, wst_ref[...], preferred_element_type=jnp.float32)
    z = z + jnp.dot(agg, wnt_ref[...], preferred_element_type=jnp.float32)
    z = jnp.maximum(z + bg_ref[...], 0.0)
    o = jnp.dot(z, wzt_ref[...], preferred_element_type=jnp.float32)
    o = o + jnp.dot(x, wxt_ref[...], preferred_element_type=jnp.float32)
    o_ref[...] = o + bo_ref[...]


@functools.partial(jax.jit, static_argnums=())
def _dense(x, p, deg, wst, wnt, bg, wzt, wxt, bo):
    return pl.pallas_call(
        _dense_body,
        out_shape=jax.ShapeDtypeStruct((N_NODES, D), jnp.float32),
    )(x, p, deg, wst, wnt, bg, wzt, wxt, bo)


def kernel(x, edge_index, W_self, W_nbr, b_gnn, W_out, b_out):
    src = edge_index[0].astype(jnp.int32)
    dst = edge_index[1].astype(jnp.int32)
    src2 = src.reshape(NW, CHUNKS_PER_W, CHUNK)
    dst2 = dst.reshape(NW, CHUNKS_PER_W, CHUNK)
    p, deg = _agg(x, src2, dst2)
    p = p.reshape(NC, N_NODES, D)
    deg = deg.reshape(NW, N_NODES)
    return _dense(
        x, p, deg,
        W_self.T, W_nbr.T, b_gnn.reshape(1, D),
        W_out[:, :D].T, W_out[:, D:].T, b_out.reshape(1, D),
    )


# speedup vs baseline: 1.3531x; 1.0159x over previous
import functools

import jax
import jax.numpy as jnp
from jax import lax
from jax.experimental import pallas as pl
from jax.experimental.pallas import tpu as pltpu
from jax.experimental.pallas import tpu_sc as plsc

N_NODES = 10000
N_EDGES = 320000
D = 128
NC = 2
NS = 16
L = 16
NW = NC * NS
E_PER_W = N_EDGES // NW
CHUNK = 80
CHUNKS_PER_W = E_PER_W // CHUNK
NBUF = 3
G = 25
NGROUPS = CHUNKS_PER_W // G
ROWS_PER_TILE = N_NODES // NS


def _zero_rows(ref, n_rows, n_cols):
    per_row = n_cols // L

    def body(i, _):
        ref[i // per_row, pl.ds((i % per_row) * L, L)] = jnp.zeros((L,), jnp.float32)
        return 0

    lax.fori_loop(0, n_rows * per_row, body, 0)


def _agg_body(x_hbm, src2_hbm, dst2_hbm, p_hbm, deg_hbm,
              agg_sh, src_ring, dst_ring, rows, deg_v, gsems, ssems, isems):
    cid = lax.axis_index("c")
    sid = lax.axis_index("s")
    wid = sid * NC + cid

    idx_src_cp = pltpu.async_copy(src2_hbm.at[wid, 0], src_ring.at[0], isems.at[0])
    idx_dst_cp = pltpu.async_copy(dst2_hbm.at[wid, 0], dst_ring.at[0], isems.at[1])

    _zero_rows(rows.at[0], CHUNK, D)

    zero_cps = []
    for t in range(ROWS_PER_TILE // CHUNK):
        zero_cps.append(pltpu.async_copy(
            rows.at[0],
            agg_sh.at[pl.ds(sid * ROWS_PER_TILE + t * CHUNK, CHUNK)],
            ssems.at[0]))
    rem = ROWS_PER_TILE % CHUNK
    if rem:
        zero_cps.append(pltpu.async_copy(
            rows.at[0, pl.ds(0, rem)],
            agg_sh.at[pl.ds(sid * ROWS_PER_TILE + (ROWS_PER_TILE // CHUNK) * CHUNK, rem)],
            ssems.at[0]))

    def zero_deg(i, _):
        deg_v[pl.ds(i * L, L)] = jnp.zeros((L,), jnp.float32)
        return 0

    lax.fori_loop(0, N_NODES // L, zero_deg, 0)

    idx_src_cp.wait()
    idx_dst_cp.wait()

    ones = jnp.full((L,), 1.0, jnp.float32)

    def gat(gb, rel, b):
        pltpu.async_copy(x_hbm.at[src_ring.at[gb, rel]], rows.at[b], gsems.at[b])

    def gat_wait(gb, rel, b):
        pltpu.make_async_copy(
            x_hbm.at[src_ring.at[gb, rel]], rows.at[b], gsems.at[b]).wait()

    def sca(gb, rel, b):
        pltpu.async_copy(rows.at[b], agg_sh.at[dst_ring.at[gb, rel]],
                         ssems.at[b], add=True)

    def sca_wait(gb, rel, b):
        pltpu.make_async_copy(rows.at[b], agg_sh.at[dst_ring.at[gb, rel]],
                              ssems.at[b]).wait()

    def deg_update(gb, rel):
        for j in range(CHUNK // L):
            idx = dst_ring[gb, rel, pl.ds(j * L, L)]
            plsc.addupdate_scatter(deg_v, [idx], ones)

    def step(g, rel, j, first=False, nxt=True):
        gb = g % 2
        b = (g + j) % NBUF
        bp = (b - 1) % NBUF
        gat_wait(gb, rel, b)
        sca(gb, rel, b)
        deg_update(gb, rel)
        if not first:
            prel = rel - 1
            if isinstance(rel, int) and rel == 0:
                sca_wait((g - 1) % 2, G - 1, bp)
            else:
                sca_wait(gb, prel, bp)
            if nxt:
                nrel = rel + 2
                if isinstance(nrel, int) and nrel >= G:
                    gat((g + 1) % 2, nrel - G, bp)
                else:
                    gat(gb, nrel, bp)

    gat(0, 1, 1)
    gat(0, 2, 2)
    for cp in zero_cps:
        cp.wait()
    gat(0, 0, 0)

    plsc.subcore_barrier()

    for g in range(NGROUPS):
        step(g, 0, 0, first=(g == 0))
        if g + 1 < NGROUPS:
            pltpu.async_copy(src2_hbm.at[wid, g + 1],
                             src_ring.at[(g + 1) % 2], isems.at[0])
            pltpu.async_copy(dst2_hbm.at[wid, g + 1],
                             dst_ring.at[(g + 1) % 2], isems.at[1])

        def body(i, _, g=g):
            for j in range(3):
                step(g, 1 + 3 * i + j, 1 + j)
            return 0

        lax.fori_loop(0, (G - 4) // 3, body, 0)

        step(g, G - 3, G - 3, nxt=True)
        if g + 1 < NGROUPS:
            pltpu.make_async_copy(src2_hbm.at[wid, g + 1],
                                  src_ring.at[(g + 1) % 2], isems.at[0]).wait()
            pltpu.make_async_copy(dst2_hbm.at[wid, g + 1],
                                  dst_ring.at[(g + 1) % 2], isems.at[1]).wait()
        step(g, G - 2, G - 2, nxt=(g + 1 < NGROUPS))
        step(g, G - 1, G - 1, nxt=(g + 1 < NGROUPS))

    sca_wait((NGROUPS - 1) % 2, G - 1, (NGROUPS - 1 + G - 1) % NBUF)

    plsc.subcore_barrier()

    row0 = sid * ROWS_PER_TILE
    pltpu.sync_copy(agg_sh.at[pl.ds(row0, ROWS_PER_TILE)],
                    p_hbm.at[cid, sid])
    pltpu.sync_copy(deg_v, deg_hbm.at[wid, 0])


_agg = pl.kernel(
    _agg_body,
    out_type=(
        jax.ShapeDtypeStruct((NC, NS, ROWS_PER_TILE, D), jnp.float32),
        jax.ShapeDtypeStruct((NW, 1, N_NODES), jnp.float32),
    ),
    mesh=plsc.VectorSubcoreMesh(core_axis_name="c", subcore_axis_name="s"),
    compiler_params=pltpu.CompilerParams(
        needs_layout_passes=False, use_tc_tiling_on_sc=False),
    scratch_types=[
        pltpu.VMEM_SHARED((N_NODES, D), jnp.float32),
        pltpu.VMEM((2, G, CHUNK), jnp.int32),
        pltpu.VMEM((2, G, CHUNK), jnp.int32),
        pltpu.VMEM((NBUF, CHUNK, D), jnp.float32),
        pltpu.VMEM((N_NODES,), jnp.float32),
        pltpu.SemaphoreType.DMA((NBUF,)),
        pltpu.SemaphoreType.DMA((NBUF,)),
        pltpu.SemaphoreType.DMA((2,)),
    ],
)


def _dense_body(x_ref, p_ref, deg_ref, wst_ref, wnt_ref, bg_ref,
                wzt_ref, wxt_ref, bo_ref, o_ref):
    x = x_ref[...]
    deg = jnp.sum(deg_ref[...], axis=0)
    scale = 1.0 / jnp.maximum(deg, 1.0)
    agg = (p_ref[0] + p_ref[1]) * scale[:, None]
    z = jnp.dot(x, wst_ref[...], preferred_element_type=jnp.float32)
    z = z + jnp.dot(agg, wnt_ref[...], preferred_element_type=jnp.float32)
    z = jnp.maximum(z + bg_ref[...], 0.0)
    o = jnp.dot(z, wzt_ref[...], preferred_element_type=jnp.float32)
    o = o + jnp.dot(x, wxt_ref[...], preferred_element_type=jnp.float32)
    o_ref[...] = o + bo_ref[...]


@functools.partial(jax.jit, static_argnums=())
def _dense(x, p, deg, wst, wnt, bg, wzt, wxt, bo):
    return pl.pallas_call(
        _dense_body,
        out_shape=jax.ShapeDtypeStruct((N_NODES, D), jnp.float32),
    )(x, p, deg, wst, wnt, bg, wzt, wxt, bo)


def kernel(x, edge_index, W_self, W_nbr, b_gnn, W_out, b_out):
    src = edge_index[0].astype(jnp.int32)
    dst = edge_index[1].astype(jnp.int32)
    src2 = src.reshape(NW, NGROUPS, G, CHUNK)
    dst2 = dst.reshape(NW, NGROUPS, G, CHUNK)
    p, deg = _agg(x, src2, dst2)
    p = p.reshape(NC, N_NODES, D)
    deg = deg.reshape(NW, N_NODES)
    return _dense(
        x, p, deg,
        W_self.T, W_nbr.T, b_gnn.reshape(1, D),
        W_out[:, :D].T, W_out[:, D:].T, b_out.reshape(1, D),
    )
